# baseline (speedup 1.0000x reference)
import functools

import jax
import jax.numpy as jnp
from jax import lax
from jax.experimental import pallas as pl
from jax.experimental.pallas import tpu as pltpu
from jax.experimental.pallas import tpu_sc as plsc

H = 128
N_NODES = 10000
N_EDGES = 320000

NC = 2
NS = 16
NW = NC * NS
L = 16

GW = 144
E_CHUNK = 80
EDGES_PER_W = N_EDGES // NW
N_CHUNKS = EDGES_PER_W // E_CHUNK
ROWS_PER_TILE = N_NODES // NS
INIT_ROWS = 25

_HIGH = jax.lax.Precision.HIGHEST


def _gelu_sc(x):
    z = x * 0.7071067811865476
    az = jnp.abs(z)
    t = 1.0 / (1.0 + 0.3275911 * az)
    poly = ((((1.061405429 * t - 1.453152027) * t + 1.421413741) * t
             - 0.284496736) * t + 0.254829592) * t
    erf_abs = 1.0 - poly * jnp.exp(-(az * az))
    erf_z = jnp.where(z < 0.0, -erf_abs, erf_abs)
    return 0.5 * x * (1.0 + erf_z)


def _stage1_body(h_ref, wa_ref, wb_ref, b1_ref, a_ref, b_ref):
    h = h_ref[...]
    a_ref[...] = jnp.dot(h, wa_ref[...], precision=_HIGH) + b1_ref[...]
    b_ref[...] = jnp.dot(h, wb_ref[...], precision=_HIGH)


def _stage1(hidden, w1a, w1b, b1e):
    blk = 1000
    grid = (N_NODES // blk,)
    return pl.pallas_call(
        _stage1_body,
        grid=grid,
        in_specs=[
            pl.BlockSpec((blk, H), lambda i: (i, 0)),
            pl.BlockSpec((H, H), lambda i: (0, 0)),
            pl.BlockSpec((H, H), lambda i: (0, 0)),
            pl.BlockSpec((1, H), lambda i: (0, 0)),
        ],
        out_specs=[
            pl.BlockSpec((blk, H), lambda i: (i, 0)),
            pl.BlockSpec((blk, H), lambda i: (i, 0)),
        ],
        out_shape=[
            jax.ShapeDtypeStruct((N_NODES, H), jnp.float32),
            jax.ShapeDtypeStruct((N_NODES, H), jnp.float32),
        ],
    )(hidden, w1a, w1b, b1e)


def _sc_body(a_hbm, b_hbm, src_hbm, dst_hbm, ew_hbm, wrow_hbm, g_hbm,
             src_v, dst_v, ew_v, a_v, b_v, g_v, wr_v, zb_v, acc_sh, sem):
    cid = lax.axis_index("core")
    sid = lax.axis_index("subcore")
    wid = sid * NC + cid
    tid = sid

    zeros16 = jnp.zeros((L,), jnp.float32)

    @pl.loop(0, INIT_ROWS)
    def _(r):
        for j in range(GW // L):
            zb_v[r, pl.ds(j * L, L)] = zeros16

    @pl.loop(0, ROWS_PER_TILE // INIT_ROWS)
    def _(c):
        pltpu.sync_copy(
            zb_v, acc_sh.at[pl.ds(tid * ROWS_PER_TILE + c * INIT_ROWS,
                                  INIT_ROWS)])

    pltpu.sync_copy(wrow_hbm, wr_v)
    e0 = jnp.where(lax.iota(jnp.int32, L) == 0, 1.0, 0.0)

    @pl.loop(0, E_CHUNK)
    def _(e):
        g_v[e, pl.ds(H, L)] = e0

    plsc.subcore_barrier()

    @pl.loop(0, N_CHUNKS)
    def _(c):
        base = wid * EDGES_PER_W + c * E_CHUNK
        pltpu.sync_copy(src_hbm.at[pl.ds(base, E_CHUNK)], src_v)
        pltpu.sync_copy(dst_hbm.at[pl.ds(base, E_CHUNK)], dst_v)
        pltpu.sync_copy(ew_hbm.at[pl.ds(base, E_CHUNK)], ew_v)
        pltpu.async_copy(a_hbm.at[src_v], a_v, sem).wait()
        pltpu.async_copy(b_hbm.at[dst_v], b_v, sem).wait()

        @pl.loop(0, E_CHUNK, step=L)
        def _(e0):
            wv = ew_v[pl.ds(e0, L)]
            for k in range(L):
                w = wv[k]
                for j in range(H // L):
                    s = pl.ds(j * L, L)
                    x = a_v[e0 + k, s] + b_v[e0 + k, s] + w * wr_v[s]
                    g_v[e0 + k, s] = _gelu_sc(x)

        pltpu.sync_copy(g_v, acc_sh.at[dst_v], add=True)

    plsc.subcore_barrier()

    pltpu.sync_copy(acc_sh.at[pl.ds(tid * ROWS_PER_TILE, ROWS_PER_TILE)],
                    g_hbm.at[cid, pl.ds(tid * ROWS_PER_TILE, ROWS_PER_TILE)])


def _stage2(a_tab, b_tab, src, dst, ew, wrow):
    mesh = plsc.VectorSubcoreMesh(core_axis_name="core",
                                  subcore_axis_name="subcore")
    kern = pl.kernel(
        _sc_body,
        out_type=jax.ShapeDtypeStruct((NC, N_NODES, GW), jnp.float32),
        mesh=mesh,
        scratch_types=[
            pltpu.VMEM((E_CHUNK,), jnp.int32),
            pltpu.VMEM((E_CHUNK,), jnp.int32),
            pltpu.VMEM((E_CHUNK,), jnp.float32),
            pltpu.VMEM((E_CHUNK, H), jnp.float32),
            pltpu.VMEM((E_CHUNK, H), jnp.float32),
            pltpu.VMEM((E_CHUNK, GW), jnp.float32),
            pltpu.VMEM((H,), jnp.float32),
            pltpu.VMEM((INIT_ROWS, GW), jnp.float32),
            pltpu.VMEM_SHARED((N_NODES, GW), jnp.float32),
            pltpu.SemaphoreType.DMA,
        ],
        compiler_params=pltpu.CompilerParams(use_tc_tiling_on_sc=False),
    )
    return kern(a_tab, b_tab, src, dst, ew, wrow)


def _stage3_body(h_ref, g_ref, w2e_ref, b2e_ref, w1h_ref, w1a_ref, b1u_ref,
                 w2u_ref, b2u_ref, gam_ref, bet_ref, o_ref):
    h = h_ref[...]
    g = g_ref[0] + g_ref[1]
    agg = (jnp.dot(g[:, :H], w2e_ref[...], precision=_HIGH)
           + g[:, H:H + 1] * b2e_ref[...])
    pre = (jnp.dot(h, w1h_ref[...], precision=_HIGH)
           + jnp.dot(agg, w1a_ref[...], precision=_HIGH) + b1u_ref[...])
    act = 0.5 * pre * (1.0 + lax.erf(pre * 0.7071067811865476))
    upd = jnp.dot(act, w2u_ref[...], precision=_HIGH) + b2u_ref[...]
    x = h + upd
    mu = jnp.mean(x, axis=-1, keepdims=True)
    var = jnp.mean((x - mu) ** 2, axis=-1, keepdims=True)
    o_ref[...] = (x - mu) / jnp.sqrt(var + 1e-5) * gam_ref[...] + bet_ref[...]


def _stage3(hidden, g, w2e, b2e, w1h, w1a, b1u, w2u, b2u, gamma, beta):
    blk = 1000
    grid = (N_NODES // blk,)
    full = lambda i: (0, 0)
    return pl.pallas_call(
        _stage3_body,
        grid=grid,
        in_specs=[
            pl.BlockSpec((blk, H), lambda i: (i, 0)),
            pl.BlockSpec((NC, blk, GW), lambda i: (0, i, 0)),
            pl.BlockSpec((H, H), full),
            pl.BlockSpec((1, H), full),
            pl.BlockSpec((H, H), full),
            pl.BlockSpec((H, H), full),
            pl.BlockSpec((1, H), full),
            pl.BlockSpec((H, H), full),
            pl.BlockSpec((1, H), full),
            pl.BlockSpec((1, H), full),
            pl.BlockSpec((1, H), full),
        ],
        out_specs=pl.BlockSpec((blk, H), lambda i: (i, 0)),
        out_shape=jax.ShapeDtypeStruct((N_NODES, H), jnp.float32),
    )(hidden, g, w2e, b2e, w1h, w1a, b1u, w2u, b2u, gamma, beta)


def kernel(hidden, edge_index, edge_weight, W1e, b1e, W2e, b2e,
           W1u, b1u, W2u, b2u, gamma, beta):
    src = edge_index[0].astype(jnp.int32)
    dst = edge_index[1].astype(jnp.int32)
    ew = edge_weight.astype(jnp.float32)

    w1a = W1e[:H]
    w1b = W1e[H:2 * H]
    wrow = W1e[2 * H]

    a_tab, b_tab = _stage1(hidden, w1a, w1b, b1e.reshape(1, H))
    g = _stage2(a_tab, b_tab, src, dst, ew, wrow)
    return _stage3(hidden, g, W2e, b2e.reshape(1, H),
                   W1u[:H], W1u[H:], b1u.reshape(1, H),
                   W2u, b2u.reshape(1, H),
                   gamma.reshape(1, H), beta.reshape(1, H))


# speedup vs baseline: 3.7090x; 3.7090x over previous
import jax
import jax.numpy as jnp
from jax import lax
from jax.experimental import pallas as pl
from jax.experimental.pallas import tpu as pltpu
from jax.experimental.pallas import tpu_sc as plsc

H = 128
N_NODES = 10000
N_EDGES = 320000

NC = 2
NS = 16
NW = NC * NS
L = 16

DW = 16
E_CHUNK = 40
EDGES_PER_W = N_EDGES // NW
N_CHUNKS = EDGES_PER_W // E_CHUNK
N_PAD = 10240
ROWS_PER_TILE = N_PAD // NS
ZCOPIES = ROWS_PER_TILE // E_CHUNK

_HIGH = jax.lax.Precision.HIGHEST


def _gelu_sc(x):
    z = x * 0.7071067811865476
    az = jnp.abs(z)
    t = 1.0 / (1.0 + 0.3275911 * az)
    poly = ((((1.061405429 * t - 1.453152027) * t + 1.421413741) * t
             - 0.284496736) * t + 0.254829592) * t
    erf_abs = 1.0 - poly * jnp.exp(-(az * az))
    erf_z = jnp.where(z < 0.0, -erf_abs, erf_abs)
    return 0.5 * x * (1.0 + erf_z)


def _stage1_body(h_ref, wa_ref, wb_ref, b1_ref, a_ref, b_ref):
    h = h_ref[...]
    a_ref[...] = jnp.dot(h, wa_ref[...], precision=_HIGH) + b1_ref[...]
    b_ref[...] = jnp.dot(h, wb_ref[...], precision=_HIGH)


def _stage1(hidden, w1a, w1b, b1e):
    blk = 1000
    grid = (N_NODES // blk,)
    return pl.pallas_call(
        _stage1_body,
        grid=grid,
        in_specs=[
            pl.BlockSpec((blk, H), lambda i: (i, 0)),
            pl.BlockSpec((H, H), lambda i: (0, 0)),
            pl.BlockSpec((H, H), lambda i: (0, 0)),
            pl.BlockSpec((1, H), lambda i: (0, 0)),
        ],
        out_specs=[
            pl.BlockSpec((blk, H), lambda i: (i, 0)),
            pl.BlockSpec((blk, H), lambda i: (i, 0)),
        ],
        out_shape=[
            jax.ShapeDtypeStruct((N_NODES, H), jnp.float32),
            jax.ShapeDtypeStruct((N_NODES, H), jnp.float32),
        ],
    )(hidden, w1a, w1b, b1e)


def _sc_body(a_hbm, b_hbm, src_hbm, dst_hbm, ew_hbm, wrow_hbm,
             g_hbm, d_hbm, srcs, dsts, ews, a2, b2, g2, ones_v, wr_v,
             acc_sh, deg_sh, si, sg, ss):
    cid = lax.axis_index("core")
    sid = lax.axis_index("subcore")
    wid = sid * NC + cid
    tid = sid

    zeros16 = jnp.zeros((L,), jnp.float32)

    @pl.loop(0, E_CHUNK)
    def _(e):
        for j in range(H // L):
            g2[0, e, pl.ds(j * L, L)] = zeros16
        ones_v[e, pl.ds(0, L)] = zeros16

    @pl.loop(0, ZCOPIES)
    def _(c):
        rows = pl.ds(tid * ROWS_PER_TILE + c * E_CHUNK, E_CHUNK)
        pltpu.sync_copy(g2.at[0], acc_sh.at[rows])
        pltpu.sync_copy(ones_v, deg_sh.at[rows])

    ones16 = jnp.ones((L,), jnp.float32)

    @pl.loop(0, E_CHUNK)
    def _(e):
        ones_v[e, pl.ds(0, L)] = ones16

    pltpu.sync_copy(wrow_hbm, wr_v)
    plsc.subcore_barrier()

    wrjs = [wr_v[pl.ds(j * L, L)] for j in range(H // L)]

    def idx_copies(k, p):
        base = pl.ds(k * E_CHUNK, E_CHUNK)
        return (
            pltpu.make_async_copy(src_hbm.at[wid, base], srcs.at[p],
                                  si.at[p]),
            pltpu.make_async_copy(dst_hbm.at[wid, base], dsts.at[p],
                                  si.at[p]),
            pltpu.make_async_copy(ew_hbm.at[wid, base],
                                  ews.at[p, pl.ds(0, E_CHUNK)], si.at[p]),
        )

    def gather_copies(b, p):
        return (
            pltpu.make_async_copy(a_hbm.at[srcs.at[p]], a2.at[b], sg.at[b]),
            pltpu.make_async_copy(b_hbm.at[dsts.at[p]], b2.at[b], sg.at[b]),
        )

    def scatter_copies(b, p):
        return (
            pltpu.make_async_copy(g2.at[b], acc_sh.at[dsts.at[p]],
                                  ss.at[b]),
            pltpu.make_async_copy(ones_v, deg_sh.at[dsts.at[p]], ss.at[b]),
        )

    for k in (0, 1):
        for c in idx_copies(k, k):
            c.start()
    for k in (0, 1):
        for c in idx_copies(k, k):
            c.wait()
        for c in gather_copies(k, k):
            c.start()

    @pl.loop(0, N_CHUNKS)
    def _(k):
        b = lax.rem(k, 2)
        p = lax.rem(k, 4)
        p2 = lax.rem(k + 2, 4)

        for c in gather_copies(b, p):
            c.wait()

        @pl.when(k >= 2)
        def _():
            for c in scatter_copies(b, p2):
                c.wait()

        @pl.when(k + 2 < N_CHUNKS)
        def _():
            for c in idx_copies(k + 2, p2):
                c.start()

        for e0, nk in ((0, L), (16, L), (32, 8)):
            wv = ews[p, pl.ds(e0, L)]
            for kk in range(nk):
                w = wv[kk]
                e = e0 + kk
                for j in range(H // L):
                    s = pl.ds(j * L, L)
                    x = a2[b, e, s] + b2[b, e, s] + w * wrjs[j]
                    g2[b, e, s] = _gelu_sc(x)

        sca, scd = scatter_copies(b, p)
        sca.start(add=True)
        scd.start(add=True)

        @pl.when(k + 2 < N_CHUNKS)
        def _():
            for c in idx_copies(k + 2, p2):
                c.wait()
            for c in gather_copies(b, p2):
                c.start()

    for k in (N_CHUNKS - 2, N_CHUNKS - 1):
        for c in scatter_copies(k % 2, k % 4):
            c.wait()

    plsc.subcore_barrier()

    rows = pl.ds(tid * ROWS_PER_TILE, ROWS_PER_TILE)
    pltpu.sync_copy(acc_sh.at[rows], g_hbm.at[cid, rows])
    pltpu.sync_copy(deg_sh.at[rows], d_hbm.at[cid, rows])


def _stage2(a_tab, b_tab, src, dst, ew, wrow):
    mesh = plsc.VectorSubcoreMesh(core_axis_name="core",
                                  subcore_axis_name="subcore")
    kern = pl.kernel(
        _sc_body,
        out_type=[
            jax.ShapeDtypeStruct((NC, N_PAD, H), jnp.float32),
            jax.ShapeDtypeStruct((NC, N_PAD, DW), jnp.float32),
        ],
        mesh=mesh,
        scratch_types=[
            pltpu.VMEM((4, E_CHUNK), jnp.int32),
            pltpu.VMEM((4, E_CHUNK), jnp.int32),
            pltpu.VMEM((4, E_CHUNK + 8), jnp.float32),
            pltpu.VMEM((2, E_CHUNK, H), jnp.float32),
            pltpu.VMEM((2, E_CHUNK, H), jnp.float32),
            pltpu.VMEM((2, E_CHUNK, H), jnp.float32),
            pltpu.VMEM((E_CHUNK, DW), jnp.float32),
            pltpu.VMEM((H,), jnp.float32),
            pltpu.VMEM_SHARED((N_PAD, H), jnp.float32),
            pltpu.VMEM_SHARED((N_PAD, DW), jnp.float32),
            pltpu.SemaphoreType.DMA((4,)),
            pltpu.SemaphoreType.DMA((2,)),
            pltpu.SemaphoreType.DMA((2,)),
        ],
        compiler_params=pltpu.CompilerParams(use_tc_tiling_on_sc=False),
    )
    srcr = src.reshape(NW, EDGES_PER_W)
    dstr = dst.reshape(NW, EDGES_PER_W)
    ewr = ew.reshape(NW, EDGES_PER_W)
    return kern(a_tab, b_tab, srcr, dstr, ewr, wrow)


def _stage3_body(h_ref, g_ref, d_ref, w2e_ref, b2e_ref, w1h_ref, w1a_ref,
                 b1u_ref, w2u_ref, b2u_ref, gam_ref, bet_ref, o_ref):
    h = h_ref[...]
    g = g_ref[0] + g_ref[1]
    deg = d_ref[0, :, :1] + d_ref[1, :, :1]
    agg = jnp.dot(g, w2e_ref[...], precision=_HIGH) + deg * b2e_ref[...]
    pre = (jnp.dot(h, w1h_ref[...], precision=_HIGH)
           + jnp.dot(agg, w1a_ref[...], precision=_HIGH) + b1u_ref[...])
    act = 0.5 * pre * (1.0 + lax.erf(pre * 0.7071067811865476))
    upd = jnp.dot(act, w2u_ref[...], precision=_HIGH) + b2u_ref[...]
    x = h + upd
    mu = jnp.mean(x, axis=-1, keepdims=True)
    var = jnp.mean((x - mu) ** 2, axis=-1, keepdims=True)
    o_ref[...] = (x - mu) / jnp.sqrt(var + 1e-5) * gam_ref[...] + bet_ref[...]


def _stage3(hidden, g, d, w2e, b2e, w1h, w1a, b1u, w2u, b2u, gamma, beta):
    blk = 1000
    grid = (N_NODES // blk,)
    full = lambda i: (0, 0)
    return pl.pallas_call(
        _stage3_body,
        grid=grid,
        in_specs=[
            pl.BlockSpec((blk, H), lambda i: (i, 0)),
            pl.BlockSpec((NC, blk, H), lambda i: (0, i, 0)),
            pl.BlockSpec((NC, blk, DW), lambda i: (0, i, 0)),
            pl.BlockSpec((H, H), full),
            pl.BlockSpec((1, H), full),
            pl.BlockSpec((H, H), full),
            pl.BlockSpec((H, H), full),
            pl.BlockSpec((1, H), full),
            pl.BlockSpec((H, H), full),
            pl.BlockSpec((1, H), full),
            pl.BlockSpec((1, H), full),
            pl.BlockSpec((1, H), full),
        ],
        out_specs=pl.BlockSpec((blk, H), lambda i: (i, 0)),
        out_shape=jax.ShapeDtypeStruct((N_NODES, H), jnp.float32),
    )(hidden, g, d, w2e, b2e, w1h, w1a, b1u, w2u, b2u, gamma, beta)


def kernel(hidden, edge_index, edge_weight, W1e, b1e, W2e, b2e,
           W1u, b1u, W2u, b2u, gamma, beta):
    src = edge_index[0].astype(jnp.int32)
    dst = edge_index[1].astype(jnp.int32)
    ew = edge_weight.astype(jnp.float32)

    w1a = W1e[:H]
    w1b = W1e[H:2 * H]
    wrow = W1e[2 * H]

    a_tab, b_tab = _stage1(hidden, w1a, w1b, b1e.reshape(1, H))
    g, d = _stage2(a_tab, b_tab, src, dst, ew, wrow)
    return _stage3(hidden, g, d, W2e, b2e.reshape(1, H),
                   W1u[:H], W1u[H:], b1u.reshape(1, H),
                   W2u, b2u.reshape(1, H),
                   gamma.reshape(1, H), beta.reshape(1, H))
